# baseline (speedup 1.0000x reference)
import functools

import jax
import jax.numpy as jnp
from jax import lax
from jax.experimental import pallas as pl
from jax.experimental.pallas import tpu as pltpu
from jax.experimental.pallas import tpu_sc as plsc

N_TASKS = 8
N_EXPERTS = 16
TOP_K = 2
W_EX = 0.01
TOKENS = 8192

NC = 1
NS = 16
NW = NC * NS
ROWS = TOKENS // NW
GROUPS = ROWS // 16


def _sc_body(logits_hbm, labels_hbm, out_hbm, logits_v, labels_v, acc_v, seg_v):
    wid = lax.axis_index("s") * NC + lax.axis_index("c")
    base = wid * ROWS
    pltpu.sync_copy(logits_hbm.at[pl.ds(base, ROWS)], logits_v)
    pltpu.sync_copy(labels_hbm.at[pl.ds(base, ROWS)], labels_v)

    zero = jnp.zeros((16,), jnp.float32)
    for r in range(16 * N_TASKS):
        acc_v[r] = zero

    lane = lax.iota(jnp.int32, 16)

    def group(j, carry):
        row = j * 16 + lane
        cols = [plsc.load_gather(logits_v, [row, jnp.full((16,), e, jnp.int32)])
                for e in range(N_EXPERTS)]
        m = cols[0]
        for e in range(1, N_EXPERTS):
            m = jnp.maximum(m, cols[e])
        exps = [jnp.exp(c - m) for c in cols]
        s = exps[0]
        for e in range(1, N_EXPERTS):
            s = s + exps[e]
        r = 1.0 / s
        t_vec = labels_v[pl.ds(j * 16, 16)]
        arow = lane * N_TASKS + t_vec
        for e in range(N_EXPERTS):
            plsc.addupdate_scatter(
                acc_v, [arow, jnp.full((16,), e, jnp.int32)], exps[e] * r)
        return carry

    lax.fori_loop(0, GROUPS, group, 0)

    for r in range(N_TASKS):
        tot = acc_v[r]
        for k in range(1, 16):
            tot = tot + acc_v[k * N_TASKS + r]
        seg_v[r] = tot
    pltpu.sync_copy(seg_v, out_hbm.at[wid])


_sc_partials = functools.partial(
    pl.kernel,
    out_type=jax.ShapeDtypeStruct((NW, N_TASKS, N_EXPERTS), jnp.float32),
    mesh=plsc.VectorSubcoreMesh(core_axis_name="c", subcore_axis_name="s",
                                num_cores=NC),
    compiler_params=pltpu.CompilerParams(needs_layout_passes=False),
    scratch_types=[
        pltpu.VMEM((ROWS, N_EXPERTS), jnp.float32),
        pltpu.VMEM((ROWS,), jnp.int32),
        pltpu.VMEM((16 * N_TASKS, N_EXPERTS), jnp.float32),
        pltpu.VMEM((N_TASKS, N_EXPERTS), jnp.float32),
    ],
)(_sc_body)


def _tc_body(part_ref, lab_ref, out_ref):
    seg = jnp.zeros((N_TASKS, N_EXPERTS), jnp.float32)
    for k in range(NW):
        seg = seg + part_ref[pl.ds(k * N_TASKS, N_TASKS), :]
    lab = lab_ref[...]
    rowid = lax.broadcasted_iota(jnp.int32, (N_TASKS, N_EXPERTS), 0)
    gate = jnp.zeros((N_TASKS, N_EXPERTS), jnp.float32)
    for t in range(N_TASKS):
        ct = jnp.sum((lab == t).astype(jnp.float32))
        gate = gate + jnp.where(rowid == t, ct, 0.0)
    ex_gate = gate * seg
    tot = jnp.sum(ex_gate) / TOP_K
    ex = ex_gate / (tot + 0.0001)
    p_ti = jnp.sum(ex, axis=1, keepdims=True) + 0.0001
    p_ei = jnp.sum(ex, axis=0, keepdims=True) + 0.0001
    expert_loss = -jnp.sum(ex * jnp.log(ex / p_ti / p_ei + 0.0001))
    out_ref[0, 0] = W_EX * expert_loss


def _tc_loss(partials, labels2d):
    return pl.pallas_call(
        _tc_body,
        out_shape=jax.ShapeDtypeStruct((1, 1), jnp.float32),
        out_specs=pl.BlockSpec(memory_space=pltpu.SMEM),
    )(partials, labels2d)


def kernel(router_logits, router_labels):
    logits = lax.stop_gradient(router_logits.astype(jnp.float32))
    labels = router_labels.astype(jnp.int32)
    partials = _sc_partials(logits, labels)
    loss = _tc_loss(partials.reshape(NW * N_TASKS, N_EXPERTS),
                    labels.reshape(64, 128))
    return loss.reshape(())


# speedup vs baseline: 1.1404x; 1.1404x over previous
import functools

import jax
import jax.numpy as jnp
from jax import lax
from jax.experimental import pallas as pl
from jax.experimental.pallas import tpu as pltpu
from jax.experimental.pallas import tpu_sc as plsc

N_TASKS = 8
N_EXPERTS = 16
TOP_K = 2
W_EX = 0.01
TOKENS = 8192

NC = 2
NS = 16
NW = NC * NS
ROWS = TOKENS // NW
GROUPS = ROWS // 16


def _sc_body(logits_hbm, labels_hbm, out_hbm, logits_v, labels_v, acc_v, seg_v):
    wid = lax.axis_index("s") * NC + lax.axis_index("c")
    base = wid * ROWS
    pltpu.sync_copy(logits_hbm.at[pl.ds(base, ROWS)], logits_v)
    pltpu.sync_copy(labels_hbm.at[pl.ds(base, ROWS)], labels_v)

    zero = jnp.zeros((16,), jnp.float32)
    for r in range(16 * N_TASKS):
        acc_v[r] = zero

    lane = lax.iota(jnp.int32, 16)

    def group(j, carry):
        row = j * 16 + lane
        cols = [plsc.load_gather(logits_v, [row, jnp.full((16,), e, jnp.int32)])
                for e in range(N_EXPERTS)]
        m = cols[0]
        for e in range(1, N_EXPERTS):
            m = jnp.maximum(m, cols[e])
        exps = [jnp.exp(c - m) for c in cols]
        s = exps[0]
        for e in range(1, N_EXPERTS):
            s = s + exps[e]
        r = 1.0 / s
        t_vec = labels_v[pl.ds(j * 16, 16)]
        arow = lane * N_TASKS + t_vec
        for e in range(N_EXPERTS):
            plsc.addupdate_scatter(
                acc_v, [arow, jnp.full((16,), e, jnp.int32)], exps[e] * r)
        return carry

    lax.fori_loop(0, GROUPS, group, 0)

    for r in range(N_TASKS):
        tot = acc_v[r]
        for k in range(1, 16):
            tot = tot + acc_v[k * N_TASKS + r]
        seg_v[r] = tot
    pltpu.sync_copy(seg_v, out_hbm.at[wid])


_sc_partials = functools.partial(
    pl.kernel,
    out_type=jax.ShapeDtypeStruct((NW, N_TASKS, N_EXPERTS), jnp.float32),
    mesh=plsc.VectorSubcoreMesh(core_axis_name="c", subcore_axis_name="s",
                                num_cores=NC),
    compiler_params=pltpu.CompilerParams(needs_layout_passes=False),
    scratch_types=[
        pltpu.VMEM((ROWS, N_EXPERTS), jnp.float32),
        pltpu.VMEM((ROWS,), jnp.int32),
        pltpu.VMEM((16 * N_TASKS, N_EXPERTS), jnp.float32),
        pltpu.VMEM((N_TASKS, N_EXPERTS), jnp.float32),
    ],
)(_sc_body)


def _tc_body(part_ref, lab_ref, out_ref):
    seg = jnp.zeros((N_TASKS, N_EXPERTS), jnp.float32)
    for k in range(NW):
        seg = seg + part_ref[pl.ds(k * N_TASKS, N_TASKS), :]
    lab = lab_ref[...]
    rowid = lax.broadcasted_iota(jnp.int32, (N_TASKS, N_EXPERTS), 0)
    gate = jnp.zeros((N_TASKS, N_EXPERTS), jnp.float32)
    for t in range(N_TASKS):
        ct = jnp.sum((lab == t).astype(jnp.float32))
        gate = gate + jnp.where(rowid == t, ct, 0.0)
    ex_gate = gate * seg
    tot = jnp.sum(ex_gate) / TOP_K
    ex = ex_gate / (tot + 0.0001)
    p_ti = jnp.sum(ex, axis=1, keepdims=True) + 0.0001
    p_ei = jnp.sum(ex, axis=0, keepdims=True) + 0.0001
    expert_loss = -jnp.sum(ex * jnp.log(ex / p_ti / p_ei + 0.0001))
    out_ref[0, 0] = W_EX * expert_loss


def _tc_loss(partials, labels2d):
    return pl.pallas_call(
        _tc_body,
        out_shape=jax.ShapeDtypeStruct((1, 1), jnp.float32),
        out_specs=pl.BlockSpec(memory_space=pltpu.SMEM),
    )(partials, labels2d)


def kernel(router_logits, router_labels):
    logits = lax.stop_gradient(router_logits.astype(jnp.float32))
    labels = router_labels.astype(jnp.int32)
    partials = _sc_partials(logits, labels)
    loss = _tc_loss(partials.reshape(NW * N_TASKS, N_EXPERTS),
                    labels.reshape(64, 128))
    return loss.reshape(())


# speedup vs baseline: 1.2919x; 1.1329x over previous
import functools

import jax
import jax.numpy as jnp
from jax import lax
from jax.experimental import pallas as pl
from jax.experimental.pallas import tpu as pltpu
from jax.experimental.pallas import tpu_sc as plsc

N_TASKS = 8
N_EXPERTS = 16
TOP_K = 2
W_EX = 0.01
TOKENS = 8192

NC = 2
NS = 16
NW = NC * NS
ROWS = TOKENS // NW
GROUPS = ROWS // 16


def _sc_body(logits_hbm, labels_hbm, out_hbm, logits_v, labels_v, acc_v, seg_v):
    wid = lax.axis_index("s") * NC + lax.axis_index("c")
    base = wid * ROWS
    pltpu.sync_copy(logits_hbm.at[:, pl.ds(base, ROWS)], logits_v)
    pltpu.sync_copy(labels_hbm.at[pl.ds(base, ROWS)], labels_v)

    zero = jnp.zeros((16,), jnp.float32)
    for r in range(16 * N_TASKS):
        acc_v[r] = zero

    lane = lax.iota(jnp.int32, 16)

    def group(j, carry):
        cols = [logits_v[e, pl.ds(j * 16, 16)] for e in range(N_EXPERTS)]
        m = cols[0]
        for e in range(1, N_EXPERTS):
            m = jnp.maximum(m, cols[e])
        exps = [jnp.exp(c - m) for c in cols]
        s = exps[0]
        for e in range(1, N_EXPERTS):
            s = s + exps[e]
        r = 1.0 / s
        t_vec = labels_v[pl.ds(j * 16, 16)]
        arow = lane * N_TASKS + t_vec
        for e in range(N_EXPERTS):
            plsc.addupdate_scatter(
                acc_v, [arow, jnp.full((16,), e, jnp.int32)], exps[e] * r)
        return carry

    lax.fori_loop(0, GROUPS, group, 0)

    for r in range(N_TASKS):
        tot = acc_v[r]
        for k in range(1, 16):
            tot = tot + acc_v[k * N_TASKS + r]
        seg_v[r] = tot
    pltpu.sync_copy(seg_v, out_hbm.at[wid])


_sc_partials = functools.partial(
    pl.kernel,
    out_type=jax.ShapeDtypeStruct((NW, N_TASKS, N_EXPERTS), jnp.float32),
    mesh=plsc.VectorSubcoreMesh(core_axis_name="c", subcore_axis_name="s",
                                num_cores=NC),
    compiler_params=pltpu.CompilerParams(needs_layout_passes=False),
    scratch_types=[
        pltpu.VMEM((N_EXPERTS, ROWS), jnp.float32),
        pltpu.VMEM((ROWS,), jnp.int32),
        pltpu.VMEM((16 * N_TASKS, N_EXPERTS), jnp.float32),
        pltpu.VMEM((N_TASKS, N_EXPERTS), jnp.float32),
    ],
)(_sc_body)


def _tc_body(part_ref, lab_ref, out_ref):
    seg = jnp.zeros((N_TASKS, N_EXPERTS), jnp.float32)
    for k in range(NW):
        seg = seg + part_ref[pl.ds(k * N_TASKS, N_TASKS), :]
    lab = lab_ref[...]
    rowid = lax.broadcasted_iota(jnp.int32, (N_TASKS, N_EXPERTS), 0)
    gate = jnp.zeros((N_TASKS, N_EXPERTS), jnp.float32)
    for t in range(N_TASKS):
        ct = jnp.sum((lab == t).astype(jnp.float32))
        gate = gate + jnp.where(rowid == t, ct, 0.0)
    ex_gate = gate * seg
    tot = jnp.sum(ex_gate) / TOP_K
    ex = ex_gate / (tot + 0.0001)
    p_ti = jnp.sum(ex, axis=1, keepdims=True) + 0.0001
    p_ei = jnp.sum(ex, axis=0, keepdims=True) + 0.0001
    expert_loss = -jnp.sum(ex * jnp.log(ex / p_ti / p_ei + 0.0001))
    out_ref[0, 0] = W_EX * expert_loss


def _tc_loss(partials, labels2d):
    return pl.pallas_call(
        _tc_body,
        out_shape=jax.ShapeDtypeStruct((1, 1), jnp.float32),
        out_specs=pl.BlockSpec(memory_space=pltpu.SMEM),
    )(partials, labels2d)


def kernel(router_logits, router_labels):
    logits = lax.stop_gradient(router_logits.astype(jnp.float32))
    labels = router_labels.astype(jnp.int32)
    partials = _sc_partials(logits.T, labels)
    loss = _tc_loss(partials.reshape(NW * N_TASKS, N_EXPERTS),
                    labels.reshape(64, 128))
    return loss.reshape(())
